# baseline (speedup 1.0000x reference)
import functools

import jax
import jax.numpy as jnp
from jax import lax
from jax.experimental import pallas as pl
from jax.experimental.pallas import tpu as pltpu
from jax.experimental.pallas import tpu_sc as plsc

N = 10000
E = 320000
D = 128

NC = 2
NS = 16
NW = NC * NS
EW = 10240
E_PAD = NW * EW

CHD = 128
NCHD = EW // CHD

CH = 64
NCHS = EW // CH
STG = 2
CPS = NCHS // STG
NB = 4

N_PAD = 10240
SLAB = N_PAD // NS
ZROWS = 8


def _zero_vec16(ref, nwords):
    def body(i, _):
        ref[pl.ds(i * 16, 16)] = jnp.zeros((16,), jnp.float32)
        return 0
    lax.fori_loop(0, nwords // 16, body, 0)


def _sc_degrees_body(src_hbm, dst_hbm, out_hbm, sidx, didx, ones_v, zbuf,
                     dsrc_sh, ddst_sh, sem):
    cid = lax.axis_index("c")
    sid = lax.axis_index("s")
    wid = sid * NC + cid

    def fill_ones(i, _):
        ones_v[pl.ds(i * 16, 16)] = jnp.ones((16,), jnp.float32)
        return 0
    lax.fori_loop(0, CHD // 16, fill_ones, 0)
    _zero_vec16(zbuf, SLAB)

    pltpu.sync_copy(zbuf, dsrc_sh.at[pl.ds(sid * SLAB, SLAB)])
    pltpu.sync_copy(zbuf, ddst_sh.at[pl.ds(sid * SLAB, SLAB)])
    pltpu.sync_copy(src_hbm.at[wid], sidx)
    pltpu.sync_copy(dst_hbm.at[wid], didx)
    plsc.subcore_barrier()

    def fire(j, _):
        pltpu.async_copy(ones_v, dsrc_sh.at[sidx.at[j]], sem, add=True)
        pltpu.async_copy(ones_v, ddst_sh.at[didx.at[j]], sem, add=True)
        return 0
    lax.fori_loop(0, NCHD, fire, 0)

    def drain(j, _):
        pltpu.make_async_copy(ones_v, dsrc_sh.at[sidx.at[0]], sem).wait()
        return 0
    lax.fori_loop(0, 2 * NCHD, drain, 0)
    plsc.subcore_barrier()

    pltpu.sync_copy(dsrc_sh.at[pl.ds(sid * SLAB, SLAB)],
                    out_hbm.at[cid, 0, pl.ds(sid * SLAB, SLAB)])
    pltpu.sync_copy(ddst_sh.at[pl.ds(sid * SLAB, SLAB)],
                    out_hbm.at[cid, 1, pl.ds(sid * SLAB, SLAB)])


def _sc_spmm_body(y_hbm, src_hbm, dst_hbm, out_hbm, srcbuf, dstbuf, rows,
                  zbuf, acc, gsem, ssem):
    cid = lax.axis_index("c")
    sid = lax.axis_index("s")
    wid = sid * NC + cid

    def zrow(i, _):
        def zcol(j, _):
            zbuf[i, pl.ds(j * 16, 16)] = jnp.zeros((16,), jnp.float32)
            return 0
        lax.fori_loop(0, D // 16, zcol, 0)
        return 0
    lax.fori_loop(0, ZROWS, zrow, 0)

    def zslab(t, _):
        pltpu.sync_copy(zbuf, acc.at[pl.ds(sid * SLAB + t * ZROWS, ZROWS)])
        return 0
    lax.fori_loop(0, SLAB // ZROWS, zslab, 0)

    def gather(j, b):
        pltpu.async_copy(y_hbm.at[srcbuf.at[pl.ds(j * CH, CH)]],
                         rows.at[b], gsem.at[b])

    def gwait(b):
        pltpu.make_async_copy(y_hbm.at[srcbuf.at[pl.ds(0, CH)]],
                              rows.at[b], gsem.at[b]).wait()

    def scat(j, b):
        pltpu.async_copy(rows.at[b], acc.at[dstbuf.at[j]], ssem.at[b],
                         add=True)

    def swait(b):
        pltpu.make_async_copy(rows.at[b], acc.at[dstbuf.at[0]],
                              ssem.at[b]).wait()

    plsc.subcore_barrier()

    for s in range(STG):
        pltpu.sync_copy(src_hbm.at[wid, pl.ds(s * CPS * CH, CPS * CH)],
                        srcbuf)
        pltpu.sync_copy(dst_hbm.at[wid, pl.ds(s * CPS, CPS)], dstbuf)

        gather(0, 0)
        gather(1, 1)

        def group(g, _):
            j0 = g * NB
            for b in range(NB):
                j = j0 + b
                bb = (b + 2) % NB

                @pl.when(j + 2 < CPS)
                def _():
                    @pl.when(j >= 2)
                    def _():
                        swait(bb)
                    gather(j + 2, bb)

                gwait(b)
                scat(j, b)
            return 0

        lax.fori_loop(0, CPS // NB, group, 0)
        for b in range(NB):
            swait(b)
    plsc.subcore_barrier()

    pltpu.sync_copy(acc.at[pl.ds(sid * SLAB, SLAB)],
                    out_hbm.at[cid, pl.ds(sid * SLAB, SLAB)])


@functools.cache
def _sc_kernels():
    mesh = plsc.VectorSubcoreMesh(core_axis_name="c", subcore_axis_name="s")
    degrees = pl.kernel(
        _sc_degrees_body,
        out_type=jax.ShapeDtypeStruct((NC, 2, N_PAD), jnp.float32),
        mesh=mesh,
        scratch_types=[
            pltpu.VMEM((NCHD, CHD), jnp.int32),
            pltpu.VMEM((NCHD, CHD), jnp.int32),
            pltpu.VMEM((CHD,), jnp.float32),
            pltpu.VMEM((SLAB,), jnp.float32),
            pltpu.VMEM_SHARED((N_PAD,), jnp.float32),
            pltpu.VMEM_SHARED((N_PAD,), jnp.float32),
            pltpu.SemaphoreType.DMA,
        ],
    )
    spmm = pl.kernel(
        _sc_spmm_body,
        out_type=jax.ShapeDtypeStruct((NC, N_PAD, D), jnp.float32),
        mesh=mesh,
        scratch_types=[
            pltpu.VMEM((CPS * CH,), jnp.int32),
            pltpu.VMEM((CPS, CH), jnp.int32),
            pltpu.VMEM((NB, CH, D), jnp.float32),
            pltpu.VMEM((ZROWS, D), jnp.float32),
            pltpu.VMEM_SHARED((N_PAD, D), jnp.float32),
            pltpu.SemaphoreType.DMA((NB,)),
            pltpu.SemaphoreType.DMA((NB,)),
        ],
    )
    return degrees, spmm


_RB = 1000
_GRID = N // _RB


def _tc_scale_matmul_body(x_ref, n_ref, w_ref, o_ref):
    o_ref[...] = jnp.dot(x_ref[...] * n_ref[...], w_ref[...],
                         preferred_element_type=jnp.float32)


def _tc_scale_matmul(x, norm_out, w):
    return pl.pallas_call(
        _tc_scale_matmul_body,
        grid=(_GRID,),
        in_specs=[
            pl.BlockSpec((_RB, D), lambda i: (i, 0)),
            pl.BlockSpec((_RB, 1), lambda i: (i, 0)),
            pl.BlockSpec((D, D), lambda i: (0, 0)),
        ],
        out_specs=pl.BlockSpec((_RB, D), lambda i: (i, 0)),
        out_shape=jax.ShapeDtypeStruct((N, D), jnp.float32),
    )(x, norm_out, w)


def _tc_mid_body(p0_ref, p1_ref, ni_ref, no_ref, b_ref, w_ref, o_ref):
    h = jnp.maximum((p0_ref[...] + p1_ref[...]) * ni_ref[...] + b_ref[...], 0.0)
    o_ref[...] = jnp.dot(h * no_ref[...], w_ref[...],
                         preferred_element_type=jnp.float32)


def _tc_mid(p0, p1, norm_in, norm_out, b, w):
    return pl.pallas_call(
        _tc_mid_body,
        grid=(_GRID,),
        in_specs=[
            pl.BlockSpec((_RB, D), lambda i: (i, 0)),
            pl.BlockSpec((_RB, D), lambda i: (i, 0)),
            pl.BlockSpec((_RB, 1), lambda i: (i, 0)),
            pl.BlockSpec((_RB, 1), lambda i: (i, 0)),
            pl.BlockSpec((1, D), lambda i: (0, 0)),
            pl.BlockSpec((D, D), lambda i: (0, 0)),
        ],
        out_specs=pl.BlockSpec((_RB, D), lambda i: (i, 0)),
        out_shape=jax.ShapeDtypeStruct((N, D), jnp.float32),
    )(p0, p1, norm_in, norm_out, b, w)


def _tc_head_body(p0_ref, p1_ref, ni_ref, b_ref, w3_ref, b3_ref, o_ref):
    i = pl.program_id(0)

    @pl.when(i == 0)
    def _():
        o_ref[...] = b3_ref[...]

    h = jnp.maximum((p0_ref[...] + p1_ref[...]) * ni_ref[...] + b_ref[...], 0.0)
    o_ref[...] += jnp.sum(jnp.dot(h, w3_ref[...],
                                  preferred_element_type=jnp.float32),
                          axis=0, keepdims=True) * (1.0 / N)


def _tc_head(p0, p1, norm_in, b, w3, b3):
    return pl.pallas_call(
        _tc_head_body,
        grid=(_GRID,),
        in_specs=[
            pl.BlockSpec((_RB, D), lambda i: (i, 0)),
            pl.BlockSpec((_RB, D), lambda i: (i, 0)),
            pl.BlockSpec((_RB, 1), lambda i: (i, 0)),
            pl.BlockSpec((1, D), lambda i: (0, 0)),
            pl.BlockSpec((D, 1), lambda i: (0, 0)),
            pl.BlockSpec((1, 1), lambda i: (0, 0)),
        ],
        out_specs=pl.BlockSpec((1, 1), lambda i: (0, 0)),
        out_shape=jax.ShapeDtypeStruct((1, 1), jnp.float32),
    )(p0, p1, norm_in, b, w3, b3)


def _norm(deg):
    return jnp.where(deg > 0, lax.rsqrt(jnp.maximum(deg, 1.0)), 0.0)


def kernel(x, edge_index, W1, b1, W2, b2, W3, b3):
    src = edge_index[0]
    dst = edge_index[1]

    pad = E_PAD - E
    pad_hi = jnp.full((pad,), N_PAD - 1, jnp.int32)
    srcA = jnp.concatenate([src, jnp.zeros((pad,), jnp.int32)])
    srcB = jnp.concatenate([src, pad_hi])
    dstP = jnp.concatenate([dst, pad_hi])
    srcA_f = srcA.reshape(NW, EW)
    dstP_s = dstP.reshape(NW, NCHS, CH)
    srcB_3 = srcB.reshape(NW, NCHD, CHD)
    dstP_3 = dstP.reshape(NW, NCHD, CHD)

    _sc_degrees, _sc_spmm = _sc_kernels()
    degp = _sc_degrees(srcB_3, dstP_3)
    deg_out = degp[0, 0, :N] + degp[1, 0, :N]
    deg_in = degp[0, 1, :N] + degp[1, 1, :N]
    norm_out = _norm(deg_out).reshape(N, 1)
    norm_in = _norm(deg_in).reshape(N, 1)

    b1r = b1.reshape(1, D)
    b2r = b2.reshape(1, D)
    b3r = b3.reshape(1, 1)

    y1 = _tc_scale_matmul(x, norm_out, W1)
    s1 = _sc_spmm(y1, srcA_f, dstP_s)
    y2 = _tc_mid(s1[0, :N], s1[1, :N], norm_in, norm_out, b1r, W2)
    s2 = _sc_spmm(y2, srcA_f, dstP_s)
    return _tc_head(s2[0, :N], s2[1, :N], norm_in, b2r, W3, b3r)


# speedup vs baseline: 3.2510x; 3.2510x over previous
import functools

import jax
import jax.numpy as jnp
from jax import lax
from jax.experimental import pallas as pl
from jax.experimental.pallas import tpu as pltpu
from jax.experimental.pallas import tpu_sc as plsc

N = 10000
E = 320000
D = 128

NC = 2
NS = 16
NW = NC * NS
EW = 10240
E_PAD = NW * EW

CHD = 128
NCHD = EW // CHD

CH = 64
NCHS = EW // CH
STG = 2
CPS = NCHS // STG
NB = 4

N_PAD = 10240
SLAB = N_PAD // NS
ZROWS = 8


def _zero_vec16(ref, nwords):
    def body(i, _):
        ref[pl.ds(i * 16, 16)] = jnp.zeros((16,), jnp.float32)
        return 0
    lax.fori_loop(0, nwords // 16, body, 0)


def _sc_degrees_body(src_hbm, dst_hbm, out_hbm, sidx, didx, ones_v, zbuf,
                     dsrc_sh, ddst_sh, sem):
    cid = lax.axis_index("c")
    sid = lax.axis_index("s")
    wid = sid * NC + cid

    def fill_ones(i, _):
        ones_v[pl.ds(i * 16, 16)] = jnp.ones((16,), jnp.float32)
        return 0
    lax.fori_loop(0, CHD // 16, fill_ones, 0)
    _zero_vec16(zbuf, SLAB)

    pltpu.sync_copy(zbuf, dsrc_sh.at[pl.ds(sid * SLAB, SLAB)])
    pltpu.sync_copy(zbuf, ddst_sh.at[pl.ds(sid * SLAB, SLAB)])
    pltpu.sync_copy(src_hbm.at[wid], sidx)
    pltpu.sync_copy(dst_hbm.at[wid], didx)
    plsc.subcore_barrier()

    def fire(j, _):
        pltpu.async_copy(ones_v, dsrc_sh.at[sidx.at[j]], sem, add=True)
        pltpu.async_copy(ones_v, ddst_sh.at[didx.at[j]], sem, add=True)
        return 0
    lax.fori_loop(0, NCHD, fire, 0)

    def drain(j, _):
        pltpu.make_async_copy(ones_v, dsrc_sh.at[sidx.at[0]], sem).wait()
        return 0
    lax.fori_loop(0, 2 * NCHD, drain, 0)
    plsc.subcore_barrier()

    pltpu.sync_copy(dsrc_sh.at[pl.ds(sid * SLAB, SLAB)],
                    out_hbm.at[cid, 0, pl.ds(sid * SLAB, SLAB)])
    pltpu.sync_copy(ddst_sh.at[pl.ds(sid * SLAB, SLAB)],
                    out_hbm.at[cid, 1, pl.ds(sid * SLAB, SLAB)])


def _sc_spmm_body(y_hbm, src_hbm, dst_hbm, out_hbm, srcbuf, dstbuf, rows,
                  zbuf, acc, gsem, ssem):
    cid = lax.axis_index("c")
    sid = lax.axis_index("s")
    wid = sid * NC + cid

    def zrow(i, _):
        def zcol(j, _):
            zbuf[i, pl.ds(j * 16, 16)] = jnp.zeros((16,), jnp.float32)
            return 0
        lax.fori_loop(0, D // 16, zcol, 0)
        return 0
    lax.fori_loop(0, ZROWS, zrow, 0)

    def zslab(t, _):
        pltpu.sync_copy(zbuf, acc.at[pl.ds(sid * SLAB + t * ZROWS, ZROWS)])
        return 0
    lax.fori_loop(0, SLAB // ZROWS, zslab, 0)

    def gather(j, b):
        pltpu.async_copy(y_hbm.at[srcbuf.at[pl.ds(j * CH, CH)]],
                         rows.at[b], gsem.at[b])

    def gwait(b):
        pltpu.make_async_copy(y_hbm.at[srcbuf.at[pl.ds(0, CH)]],
                              rows.at[b], gsem.at[b]).wait()

    def scat(j, b):
        pltpu.async_copy(rows.at[b], acc.at[dstbuf.at[j]], ssem.at[b],
                         add=True)

    def swait(b):
        pltpu.make_async_copy(rows.at[b], acc.at[dstbuf.at[0]],
                              ssem.at[b]).wait()

    plsc.subcore_barrier()

    for s in range(STG):
        pltpu.sync_copy(src_hbm.at[wid, pl.ds(s * CPS * CH, CPS * CH)],
                        srcbuf)
        pltpu.sync_copy(dst_hbm.at[wid, pl.ds(s * CPS, CPS)], dstbuf)

        gather(0, 0)
        gather(1, 1)

        def group(g, _):
            j0 = g * NB
            for b in range(NB):
                j = j0 + b
                bb = (b + 2) % NB

                @pl.when(j + 2 < CPS)
                def _():
                    @pl.when(j >= 2)
                    def _():
                        swait(bb)
                    gather(j + 2, bb)

                gwait(b)
                scat(j, b)
            return 0

        lax.fori_loop(0, CPS // NB, group, 0)
        for b in range(NB):
            swait(b)
    plsc.subcore_barrier()

    pltpu.sync_copy(acc.at[pl.ds(sid * SLAB, SLAB)],
                    out_hbm.at[cid, pl.ds(sid * SLAB, SLAB)])


@functools.cache
def _sc_kernels():
    mesh = plsc.VectorSubcoreMesh(core_axis_name="c", subcore_axis_name="s")
    degrees = pl.kernel(
        _sc_degrees_body,
        out_type=jax.ShapeDtypeStruct((NC, 2, N_PAD), jnp.float32),
        mesh=mesh,
        scratch_types=[
            pltpu.VMEM((NCHD, CHD), jnp.int32),
            pltpu.VMEM((NCHD, CHD), jnp.int32),
            pltpu.VMEM((CHD,), jnp.float32),
            pltpu.VMEM((SLAB,), jnp.float32),
            pltpu.VMEM_SHARED((N_PAD,), jnp.float32),
            pltpu.VMEM_SHARED((N_PAD,), jnp.float32),
            pltpu.SemaphoreType.DMA,
        ],
    )
    spmm = pl.kernel(
        _sc_spmm_body,
        out_type=jax.ShapeDtypeStruct((NC, N_PAD, D), jnp.float32),
        mesh=mesh,
        scratch_types=[
            pltpu.VMEM((CPS * CH,), jnp.int32),
            pltpu.VMEM((CPS, CH), jnp.int32),
            pltpu.VMEM((NB, CH, D), jnp.float32),
            pltpu.VMEM((ZROWS, D), jnp.float32),
            pltpu.VMEM_SHARED((N_PAD, D), jnp.float32),
            pltpu.SemaphoreType.DMA((NB,)),
            pltpu.SemaphoreType.DMA((NB,)),
        ],
    )
    return degrees, spmm


_RB = 1000
_GRID = N // _RB


def _tc_scale_matmul_body(x_ref, n_ref, w_ref, o_ref):
    o_ref[...] = jnp.dot(x_ref[...] * n_ref[...], w_ref[...],
                         preferred_element_type=jnp.float32)


def _tc_scale_matmul(x, norm_out, w):
    return pl.pallas_call(
        _tc_scale_matmul_body,
        grid=(_GRID,),
        in_specs=[
            pl.BlockSpec((_RB, D), lambda i: (i, 0)),
            pl.BlockSpec((_RB, 1), lambda i: (i, 0)),
            pl.BlockSpec((D, D), lambda i: (0, 0)),
        ],
        out_specs=pl.BlockSpec((_RB, D), lambda i: (i, 0)),
        out_shape=jax.ShapeDtypeStruct((N, D), jnp.float32),
    )(x, norm_out, w)


def _tc_mid_body(p0_ref, p1_ref, ni_ref, no_ref, b_ref, w_ref, o_ref):
    h = jnp.maximum((p0_ref[...] + p1_ref[...]) * ni_ref[...] + b_ref[...], 0.0)
    o_ref[...] = jnp.dot(h * no_ref[...], w_ref[...],
                         preferred_element_type=jnp.float32)


def _tc_mid(p0, p1, norm_in, norm_out, b, w):
    return pl.pallas_call(
        _tc_mid_body,
        grid=(_GRID,),
        in_specs=[
            pl.BlockSpec((_RB, D), lambda i: (i, 0)),
            pl.BlockSpec((_RB, D), lambda i: (i, 0)),
            pl.BlockSpec((_RB, 1), lambda i: (i, 0)),
            pl.BlockSpec((_RB, 1), lambda i: (i, 0)),
            pl.BlockSpec((1, D), lambda i: (0, 0)),
            pl.BlockSpec((D, D), lambda i: (0, 0)),
        ],
        out_specs=pl.BlockSpec((_RB, D), lambda i: (i, 0)),
        out_shape=jax.ShapeDtypeStruct((N, D), jnp.float32),
    )(p0, p1, norm_in, norm_out, b, w)


def _tc_head_body(p0_ref, p1_ref, ni_ref, b_ref, w3_ref, b3_ref, o_ref):
    i = pl.program_id(0)

    @pl.when(i == 0)
    def _():
        o_ref[...] = b3_ref[...]

    h = jnp.maximum((p0_ref[...] + p1_ref[...]) * ni_ref[...] + b_ref[...], 0.0)
    o_ref[...] += jnp.sum(jnp.dot(h, w3_ref[...],
                                  preferred_element_type=jnp.float32),
                          axis=0, keepdims=True) * (1.0 / N)


def _tc_head(p0, p1, norm_in, b, w3, b3):
    return pl.pallas_call(
        _tc_head_body,
        grid=(_GRID,),
        in_specs=[
            pl.BlockSpec((_RB, D), lambda i: (i, 0)),
            pl.BlockSpec((_RB, D), lambda i: (i, 0)),
            pl.BlockSpec((_RB, 1), lambda i: (i, 0)),
            pl.BlockSpec((1, D), lambda i: (0, 0)),
            pl.BlockSpec((D, 1), lambda i: (0, 0)),
            pl.BlockSpec((1, 1), lambda i: (0, 0)),
        ],
        out_specs=pl.BlockSpec((1, 1), lambda i: (0, 0)),
        out_shape=jax.ShapeDtypeStruct((1, 1), jnp.float32),
    )(p0, p1, norm_in, b, w3, b3)


def _norm(deg):
    return jnp.where(deg > 0, lax.rsqrt(jnp.maximum(deg, 1.0)), 0.0)


def kernel(x, edge_index, W1, b1, W2, b2, W3, b3):
    src = edge_index[0]
    dst = edge_index[1]

    pad = E_PAD - E
    ar = lax.iota(jnp.int32, pad)
    pad_hi = N + (ar % (N_PAD - N))
    srcA = jnp.concatenate([src, ar % 512])
    srcB = jnp.concatenate([src, pad_hi])
    dstP = jnp.concatenate([dst, pad_hi])
    srcA_f = srcA.reshape(NW, EW)
    dstP_s = dstP.reshape(NW, NCHS, CH)
    srcB_3 = srcB.reshape(NW, NCHD, CHD)
    dstP_3 = dstP.reshape(NW, NCHD, CHD)

    _sc_degrees, _sc_spmm = _sc_kernels()
    degp = _sc_degrees(srcB_3, dstP_3)
    deg_out = degp[0, 0, :N] + degp[1, 0, :N]
    deg_in = degp[0, 1, :N] + degp[1, 1, :N]
    norm_out = _norm(deg_out).reshape(N, 1)
    norm_in = _norm(deg_in).reshape(N, 1)

    b1r = b1.reshape(1, D)
    b2r = b2.reshape(1, D)
    b3r = b3.reshape(1, 1)

    y1 = _tc_scale_matmul(x, norm_out, W1)
    s1 = _sc_spmm(y1, srcA_f, dstP_s)
    y2 = _tc_mid(s1[0, :N], s1[1, :N], norm_in, norm_out, b1r, W2)
    s2 = _sc_spmm(y2, srcA_f, dstP_s)
    return _tc_head(s2[0, :N], s2[1, :N], norm_in, b2r, W3, b3r)
